# baseline (speedup 1.0000x reference)
import functools

import jax
import jax.numpy as jnp
from jax import lax
from jax.experimental import pallas as pl
from jax.experimental.pallas import tpu as pltpu
from jax.experimental.pallas import tpu_sc as plsc

def _i0(_=None):
    return jnp.int32(0)


E = 320000
N_NODES = 10000
D = 128
R = 16
H = 256

_EH = E // 2
_BE = 2000
_GRID_A = _EH // _BE


def _dot3(a, b):
    ah = a.astype(jnp.bfloat16)
    al = (a - ah.astype(jnp.float32)).astype(jnp.bfloat16)
    bh = b.astype(jnp.bfloat16)
    bl = (b - bh.astype(jnp.float32)).astype(jnp.bfloat16)
    d = lambda x, y: jnp.dot(x, y, preferred_element_type=jnp.float32)
    return d(ah, bh) + (d(ah, bl) + d(al, bh))


def _edge_body(msg_ref, rbf_ref, w_ref, out_ref):
    t = _dot3(rbf_ref[...], w_ref[...])
    out_ref[...] = msg_ref[...] * t


def _edge_stage(messages, rbf, W_rbf, h):
    off = h * _GRID_A
    return pl.pallas_call(
        _edge_body,
        name="edge_stage",
        grid=(_GRID_A,),
        in_specs=[
            pl.BlockSpec((_BE, D), lambda i: (i + off, _i0())),
            pl.BlockSpec((_BE, R), lambda i: (i + off, _i0())),
            pl.BlockSpec((R, D), lambda i: (_i0(), _i0())),
        ],
        out_specs=pl.BlockSpec((_BE, D), lambda i: (i, _i0())),
        out_shape=jax.ShapeDtypeStruct((_EH, D), jnp.float32),
    )(messages, rbf, W_rbf)


_NC = 2
_NS = 16
_NW = _NC * _NS
_EPW = _EH // _NW
_CH = 40
_SCH = 40
_KSC = _SCH // _CH
_NSC = _EPW // _SCH
_NBUF = 3
_NPT = (N_NODES // _NS) // 8 * 8
_TAIL0 = _NS * _NPT
_TAIL = N_NODES - _TAIL0


def _sc_body(m_hbm, idx_hbm, zero_hbm, out_hbm, rows_v, idx_v, acc_sh,
             sem_r, sem_i, sem_s):
    cid = lax.axis_index("c")
    sid = lax.axis_index("s")
    base = (cid * jnp.int32(_NS) + sid) * jnp.int32(_EPW)
    row0 = sid * jnp.int32(_NPT)

    def issue(t, b):
        start = base + t * jnp.int32(_SCH)
        pltpu.async_copy(m_hbm.at[pl.ds(start, _SCH)], rows_v.at[b], sem_r)
        pltpu.async_copy(idx_hbm.at[pl.ds(start, _CH)], idx_v.at[b, jnp.int32(0)],
                         sem_i)

    def wait_in(t, b):
        start = base + t * jnp.int32(_SCH)
        pltpu.make_async_copy(m_hbm.at[pl.ds(start, _SCH)], rows_v.at[b],
                              sem_r).wait()
        pltpu.make_async_copy(idx_hbm.at[pl.ds(start, _CH)],
                              idx_v.at[b, jnp.int32(0)], sem_i).wait()

    def wait_scatter(b):
        pltpu.make_async_copy(rows_v.at[b],
                              acc_sh.at[idx_v.at[b, jnp.int32(0)]],
                              sem_s).wait()

    issue(jnp.int32(0), jnp.int32(0))
    pltpu.sync_copy(zero_hbm.at[pl.ds(row0, _NPT)], acc_sh.at[pl.ds(row0, _NPT)])

    @pl.when(sid == 0)
    def _():
        pltpu.sync_copy(zero_hbm.at[pl.ds(_TAIL0, _TAIL)],
                        acc_sh.at[pl.ds(_TAIL0, _TAIL)])

    plsc.subcore_barrier()
    issue(jnp.int32(1), jnp.int32(1))

    @pl.loop(jnp.int32(0), jnp.int32(_NSC))
    def _(t):
        b = lax.rem(t, jnp.int32(_NBUF))
        wait_in(t, b)
        pltpu.async_copy(rows_v.at[b], acc_sh.at[idx_v.at[b, jnp.int32(0)]],
                         sem_s, add=True)

        @pl.when(t + jnp.int32(2) < jnp.int32(_NSC))
        def _():
            nb = lax.rem(t + jnp.int32(2), jnp.int32(_NBUF))

            @pl.when(t >= jnp.int32(1))
            def _():
                wait_scatter(nb)

            issue(t + jnp.int32(2), nb)

    for tt in (_NSC - 3, _NSC - 2, _NSC - 1):
        wait_scatter(jnp.int32(tt % _NBUF))
    plsc.subcore_barrier()
    pltpu.sync_copy(acc_sh.at[pl.ds(row0, _NPT)],
                    out_hbm.at[cid, pl.ds(row0, _NPT)])

    @pl.when(sid == 0)
    def _():
        pltpu.sync_copy(acc_sh.at[pl.ds(_TAIL0, _TAIL)],
                        out_hbm.at[cid, pl.ds(_TAIL0, _TAIL)])


def _sc_segment(m, idx32, zeros):
    mesh = plsc.VectorSubcoreMesh(core_axis_name="c", subcore_axis_name="s")
    run = functools.partial(
        pl.kernel,
        mesh=mesh,
        out_type=jax.ShapeDtypeStruct((_NC, N_NODES, D), jnp.float32),
        scratch_types=[
            pltpu.VMEM((_NBUF, _SCH, D), jnp.float32),
            pltpu.VMEM((_NBUF, _KSC, _CH), jnp.int32),
            pltpu.VMEM_SHARED((N_NODES, D), jnp.float32),
            pltpu.SemaphoreType.DMA,
            pltpu.SemaphoreType.DMA,
            pltpu.SemaphoreType.DMA,
        ],
    )(_sc_body)
    return run(m, idx32, zeros)


_BN = 1000
_GRID_C = N_NODES // _BN


def _mlp_body(p0_ref, p1_ref, wup_ref, w0_ref, b0_ref, w1_ref, b1_ref,
              w2_ref, b2_ref, wf_ref, out_ref):
    s = (p0_ref[0] + p0_ref[1]) + (p1_ref[0] + p1_ref[1])
    up = _dot3(s, wup_ref[...])
    for w, b in ((w0_ref, b0_ref), (w1_ref, b1_ref), (w2_ref, b2_ref)):
        up = _dot3(up, w[...]) + b[...]
        up = up * jax.nn.sigmoid(up)
    out_ref[...] = _dot3(up, wf_ref[...])


def _mlp_stage(p0, p1, W_up, W_d0, b_d0, W_d1, b_d1, W_d2, b_d2, W_final):
    wspec = lambda shape: pl.BlockSpec(shape, lambda i: (_i0(),) * len(shape))
    return pl.pallas_call(
        _mlp_body,
        name="mlp_stage",
        grid=(_GRID_C,),
        in_specs=[
            pl.BlockSpec((_NC, _BN, D), lambda i: (_i0(), i, _i0())),
            pl.BlockSpec((_NC, _BN, D), lambda i: (_i0(), i, _i0())),
            wspec((D, H)),
            wspec((H, H)), wspec((1, H)),
            wspec((H, H)), wspec((1, H)),
            wspec((H, H)), wspec((1, H)),
            wspec((H, 1)),
        ],
        out_specs=pl.BlockSpec((_BN, 1), lambda i: (i, _i0())),
        out_shape=jax.ShapeDtypeStruct((N_NODES, 1), jnp.float32),
    )(p0, p1, W_up, W_d0, b_d0, W_d1, b_d1, W_d2, b_d2, W_final)


def kernel(messages, rbf, idx_i, n_particles, W_rbf, W_up,
           W_d0, b_d0, W_d1, b_d1, W_d2, b_d2, W_final):
    del n_particles
    out_dtype = jnp.result_type(jnp.float32, W_up.dtype)
    f32 = lambda x: x.astype(jnp.float32)
    idx32 = idx_i.astype(jnp.int32)
    zeros = jnp.zeros((N_NODES, D), jnp.float32)
    msgs, rbf32, wrbf = f32(messages), f32(rbf), f32(W_rbf)
    m0 = _edge_stage(msgs, rbf32, wrbf, 0)
    p0 = _sc_segment(m0, idx32[:_EH], zeros)
    m1 = _edge_stage(msgs, rbf32, wrbf, 1)
    p1 = _sc_segment(m1, idx32[_EH:], zeros)
    out = _mlp_stage(p0, p1, f32(W_up), f32(W_d0), f32(b_d0).reshape(1, H),
                     f32(W_d1), f32(b_d1).reshape(1, H),
                     f32(W_d2), f32(b_d2).reshape(1, H), f32(W_final))
    return out.astype(out_dtype)


# speedup vs baseline: 1.0171x; 1.0171x over previous
import functools

import jax
import jax.numpy as jnp
from jax import lax
from jax.experimental import pallas as pl
from jax.experimental.pallas import tpu as pltpu
from jax.experimental.pallas import tpu_sc as plsc

def _i0(_=None):
    return jnp.int32(0)


E = 320000
N_NODES = 10000
D = 128
R = 16
H = 256

_BE = 2560
_GRID_A = E // _BE


def _dot3(a, b):
    ah = a.astype(jnp.bfloat16)
    al = (a - ah.astype(jnp.float32)).astype(jnp.bfloat16)
    bh = b.astype(jnp.bfloat16)
    bl = (b - bh.astype(jnp.float32)).astype(jnp.bfloat16)
    d = lambda x, y: jnp.dot(x, y, preferred_element_type=jnp.float32)
    return d(ah, bh) + (d(ah, bl) + d(al, bh))


def _edge_body(msg_ref, rbf_ref, w_ref, out_ref):
    t = _dot3(rbf_ref[...], w_ref[...])
    out_ref[...] = msg_ref[...] * t


def _edge_stage(messages, rbf, W_rbf):
    return pl.pallas_call(
        _edge_body,
        name="edge_stage",
        grid=(_GRID_A,),
        in_specs=[
            pl.BlockSpec((_BE, D), lambda i: (i, _i0())),
            pl.BlockSpec((_BE, R), lambda i: (i, _i0())),
            pl.BlockSpec((R, D), lambda i: (_i0(), _i0())),
        ],
        out_specs=pl.BlockSpec((_BE, D), lambda i: (i, _i0())),
        out_shape=jax.ShapeDtypeStruct((E, D), jnp.float32),
    )(messages, rbf, W_rbf)


_NC = 2
_NS = 16
_NW = _NC * _NS
_EPW = E // _NW
_CH = 128
_SCH = 128
_KSC = _SCH // _CH
_NSC = _EPW // _SCH
_TAIL_E = _EPW - _NSC * _SCH
_NBUF = 2
_NPT = (N_NODES // _NS) // 8 * 8
_TAIL0 = _NS * _NPT
_TAIL = N_NODES - _TAIL0


def _sc_body(m_hbm, idx_hbm, zero_hbm, out_hbm, rows_v, idx_v, rows_t, idx_t,
             acc_sh, sem_r, sem_i, sem_s):
    cid = lax.axis_index("c")
    sid = lax.axis_index("s")
    base = (cid * jnp.int32(_NS) + sid) * jnp.int32(_EPW)
    row0 = sid * jnp.int32(_NPT)

    def issue(t, b):
        start = base + t * jnp.int32(_SCH)
        pltpu.async_copy(m_hbm.at[pl.ds(start, _SCH)], rows_v.at[b], sem_r)
        pltpu.async_copy(idx_hbm.at[pl.ds(start, _CH)], idx_v.at[b, jnp.int32(0)],
                         sem_i)

    def wait_in(t, b):
        start = base + t * jnp.int32(_SCH)
        pltpu.make_async_copy(m_hbm.at[pl.ds(start, _SCH)], rows_v.at[b],
                              sem_r).wait()
        pltpu.make_async_copy(idx_hbm.at[pl.ds(start, _CH)],
                              idx_v.at[b, jnp.int32(0)], sem_i).wait()

    def wait_scatter(b):
        pltpu.make_async_copy(rows_v.at[b],
                              acc_sh.at[idx_v.at[b, jnp.int32(0)]],
                              sem_s).wait()

    issue(jnp.int32(0), jnp.int32(0))
    pltpu.sync_copy(zero_hbm.at[pl.ds(row0, _NPT)], acc_sh.at[pl.ds(row0, _NPT)])

    @pl.when(sid == 0)
    def _():
        pltpu.sync_copy(zero_hbm.at[pl.ds(_TAIL0, _TAIL)],
                        acc_sh.at[pl.ds(_TAIL0, _TAIL)])

    plsc.subcore_barrier()

    @pl.loop(jnp.int32(0), jnp.int32(_NSC))
    def _(t):
        b = lax.rem(t, jnp.int32(_NBUF))
        wait_in(t, b)
        pltpu.async_copy(rows_v.at[b], acc_sh.at[idx_v.at[b, jnp.int32(0)]],
                         sem_s, add=True)

        @pl.when(t + jnp.int32(1) < jnp.int32(_NSC))
        def _():
            nb = lax.rem(t + jnp.int32(1), jnp.int32(_NBUF))

            @pl.when(t >= jnp.int32(1))
            def _():
                wait_scatter(nb)

            issue(t + jnp.int32(1), nb)

    for tt in (_NSC - 2, _NSC - 1):
        wait_scatter(jnp.int32(tt % _NBUF))
    tstart = base + jnp.int32(_NSC * _SCH)
    pltpu.sync_copy(m_hbm.at[pl.ds(tstart, _TAIL_E)], rows_t)
    pltpu.sync_copy(idx_hbm.at[pl.ds(tstart, _TAIL_E)], idx_t.at[jnp.int32(0)])
    pltpu.sync_copy(rows_t, acc_sh.at[idx_t.at[jnp.int32(0)]], add=True)
    plsc.subcore_barrier()
    pltpu.sync_copy(acc_sh.at[pl.ds(row0, _NPT)],
                    out_hbm.at[cid, pl.ds(row0, _NPT)])

    @pl.when(sid == 0)
    def _():
        pltpu.sync_copy(acc_sh.at[pl.ds(_TAIL0, _TAIL)],
                        out_hbm.at[cid, pl.ds(_TAIL0, _TAIL)])


def _sc_segment(m, idx32, zeros):
    mesh = plsc.VectorSubcoreMesh(core_axis_name="c", subcore_axis_name="s")
    run = functools.partial(
        pl.kernel,
        mesh=mesh,
        out_type=jax.ShapeDtypeStruct((_NC, N_NODES, D), jnp.float32),
        scratch_types=[
            pltpu.VMEM((_NBUF, _SCH, D), jnp.float32),
            pltpu.VMEM((_NBUF, _KSC, _CH), jnp.int32),
            pltpu.VMEM((_TAIL_E, D), jnp.float32),
            pltpu.VMEM((1, _TAIL_E), jnp.int32),
            pltpu.VMEM_SHARED((N_NODES, D), jnp.float32),
            pltpu.SemaphoreType.DMA,
            pltpu.SemaphoreType.DMA,
            pltpu.SemaphoreType.DMA,
        ],
    )(_sc_body)
    return run(m, idx32, zeros)


_BN = 1000
_GRID_C = N_NODES // _BN


def _mlp_body(part_ref, wup_ref, w0_ref, b0_ref, w1_ref, b1_ref,
              w2_ref, b2_ref, wf_ref, out_ref):
    s = part_ref[0] + part_ref[1]
    up = _dot3(s, wup_ref[...])
    for w, b in ((w0_ref, b0_ref), (w1_ref, b1_ref), (w2_ref, b2_ref)):
        up = _dot3(up, w[...]) + b[...]
        up = up * jax.nn.sigmoid(up)
    out_ref[...] = _dot3(up, wf_ref[...])


def _mlp_stage(partials, W_up, W_d0, b_d0, W_d1, b_d1, W_d2, b_d2, W_final):
    wspec = lambda shape: pl.BlockSpec(shape, lambda i: (_i0(),) * len(shape))
    return pl.pallas_call(
        _mlp_body,
        name="mlp_stage",
        grid=(_GRID_C,),
        in_specs=[
            pl.BlockSpec((_NC, _BN, D), lambda i: (_i0(), i, _i0())),
            wspec((D, H)),
            wspec((H, H)), wspec((1, H)),
            wspec((H, H)), wspec((1, H)),
            wspec((H, H)), wspec((1, H)),
            wspec((H, 1)),
        ],
        out_specs=pl.BlockSpec((_BN, 1), lambda i: (i, _i0())),
        out_shape=jax.ShapeDtypeStruct((N_NODES, 1), jnp.float32),
    )(partials, W_up, W_d0, b_d0, W_d1, b_d1, W_d2, b_d2, W_final)


def kernel(messages, rbf, idx_i, n_particles, W_rbf, W_up,
           W_d0, b_d0, W_d1, b_d1, W_d2, b_d2, W_final):
    del n_particles
    out_dtype = jnp.result_type(jnp.float32, W_up.dtype)
    f32 = lambda x: x.astype(jnp.float32)
    idx32 = idx_i.astype(jnp.int32)
    m = _edge_stage(f32(messages), f32(rbf), f32(W_rbf))
    zeros = jnp.zeros((N_NODES, D), jnp.float32)
    partials = _sc_segment(m, idx32, zeros)
    out = _mlp_stage(partials, f32(W_up), f32(W_d0), f32(b_d0).reshape(1, H),
                     f32(W_d1), f32(b_d1).reshape(1, H),
                     f32(W_d2), f32(b_d2).reshape(1, H), f32(W_final))
    return out.astype(out_dtype)


# speedup vs baseline: 1.0659x; 1.0480x over previous
import functools

import jax
import jax.numpy as jnp
from jax import lax
from jax.experimental import pallas as pl
from jax.experimental.pallas import tpu as pltpu
from jax.experimental.pallas import tpu_sc as plsc

def _i0(_=None):
    return jnp.int32(0)


E = 320000
N_NODES = 10000
D = 128
R = 16
H = 256

_BE = 2560
_GRID_A = E // _BE


def _dot3(a, b):
    ah = a.astype(jnp.bfloat16)
    al = (a - ah.astype(jnp.float32)).astype(jnp.bfloat16)
    bh = b.astype(jnp.bfloat16)
    bl = (b - bh.astype(jnp.float32)).astype(jnp.bfloat16)
    d = lambda x, y: jnp.dot(x, y, preferred_element_type=jnp.float32)
    return d(ah, bh) + (d(ah, bl) + d(al, bh))


def _edge_body(msg_ref, rbf_ref, w_ref, out_ref):
    t = _dot3(rbf_ref[...], w_ref[...])
    out_ref[...] = msg_ref[...] * t


def _edge_stage(messages, rbf, W_rbf):
    return pl.pallas_call(
        _edge_body,
        name="edge_stage",
        grid=(_GRID_A,),
        in_specs=[
            pl.BlockSpec((_BE, D), lambda i: (i, _i0())),
            pl.BlockSpec((_BE, R), lambda i: (i, _i0())),
            pl.BlockSpec((R, D), lambda i: (_i0(), _i0())),
        ],
        out_specs=pl.BlockSpec((_BE, D), lambda i: (i, _i0())),
        out_shape=jax.ShapeDtypeStruct((E, D), jnp.float32),
    )(messages, rbf, W_rbf)


_NC = 2
_NS = 16
_NW = _NC * _NS
_EPW = E // _NW
_CH = 104
_SCH = 104
_KSC = _SCH // _CH
_NSC = _EPW // _SCH
_TAIL_E = _EPW - _NSC * _SCH
_NBUF = 3
_NPT = (N_NODES // _NS) // 8 * 8
_TAIL0 = _NS * _NPT
_TAIL = N_NODES - _TAIL0


def _sc_body(m_hbm, idx_hbm, zero_hbm, out_hbm, rows_v, idx_v, rows_t, idx_t,
             acc_sh, sem_r, sem_i, sem_s):
    cid = lax.axis_index("c")
    sid = lax.axis_index("s")
    base = (cid * jnp.int32(_NS) + sid) * jnp.int32(_EPW)
    row0 = sid * jnp.int32(_NPT)

    def issue(t, b):
        start = base + t * jnp.int32(_SCH)
        pltpu.async_copy(m_hbm.at[pl.ds(start, _SCH)], rows_v.at[b], sem_r)
        pltpu.async_copy(idx_hbm.at[pl.ds(start, _CH)], idx_v.at[b, jnp.int32(0)],
                         sem_i)

    def wait_in(t, b):
        start = base + t * jnp.int32(_SCH)
        pltpu.make_async_copy(m_hbm.at[pl.ds(start, _SCH)], rows_v.at[b],
                              sem_r).wait()
        pltpu.make_async_copy(idx_hbm.at[pl.ds(start, _CH)],
                              idx_v.at[b, jnp.int32(0)], sem_i).wait()

    def wait_scatter(b):
        pltpu.make_async_copy(rows_v.at[b],
                              acc_sh.at[idx_v.at[b, jnp.int32(0)]],
                              sem_s).wait()

    issue(jnp.int32(0), jnp.int32(0))
    pltpu.sync_copy(zero_hbm.at[pl.ds(row0, _NPT)], acc_sh.at[pl.ds(row0, _NPT)])

    @pl.when(sid == 0)
    def _():
        pltpu.sync_copy(zero_hbm.at[pl.ds(_TAIL0, _TAIL)],
                        acc_sh.at[pl.ds(_TAIL0, _TAIL)])

    plsc.subcore_barrier()
    issue(jnp.int32(1), jnp.int32(1))

    @pl.loop(jnp.int32(0), jnp.int32(_NSC))
    def _(t):
        b = lax.rem(t, jnp.int32(_NBUF))
        wait_in(t, b)
        pltpu.async_copy(rows_v.at[b], acc_sh.at[idx_v.at[b, jnp.int32(0)]],
                         sem_s, add=True)

        @pl.when(t + jnp.int32(2) < jnp.int32(_NSC))
        def _():
            nb = lax.rem(t + jnp.int32(2), jnp.int32(_NBUF))

            @pl.when(t >= jnp.int32(1))
            def _():
                wait_scatter(nb)

            issue(t + jnp.int32(2), nb)

    for tt in (_NSC - 3, _NSC - 2, _NSC - 1):
        wait_scatter(jnp.int32(tt % _NBUF))
    tstart = base + jnp.int32(_NSC * _SCH)
    pltpu.sync_copy(m_hbm.at[pl.ds(tstart, _TAIL_E)], rows_t)
    pltpu.sync_copy(idx_hbm.at[pl.ds(tstart, _TAIL_E)], idx_t.at[jnp.int32(0)])
    pltpu.sync_copy(rows_t, acc_sh.at[idx_t.at[jnp.int32(0)]], add=True)
    plsc.subcore_barrier()
    pltpu.sync_copy(acc_sh.at[pl.ds(row0, _NPT)],
                    out_hbm.at[cid, pl.ds(row0, _NPT)])

    @pl.when(sid == 0)
    def _():
        pltpu.sync_copy(acc_sh.at[pl.ds(_TAIL0, _TAIL)],
                        out_hbm.at[cid, pl.ds(_TAIL0, _TAIL)])


def _sc_segment(m, idx32, zeros):
    mesh = plsc.VectorSubcoreMesh(core_axis_name="c", subcore_axis_name="s")
    run = functools.partial(
        pl.kernel,
        mesh=mesh,
        out_type=jax.ShapeDtypeStruct((_NC, N_NODES, D), jnp.float32),
        scratch_types=[
            pltpu.VMEM((_NBUF, _SCH, D), jnp.float32),
            pltpu.VMEM((_NBUF, _KSC, _CH), jnp.int32),
            pltpu.VMEM((_TAIL_E, D), jnp.float32),
            pltpu.VMEM((1, _TAIL_E), jnp.int32),
            pltpu.VMEM_SHARED((N_NODES, D), jnp.float32),
            pltpu.SemaphoreType.DMA,
            pltpu.SemaphoreType.DMA,
            pltpu.SemaphoreType.DMA,
        ],
    )(_sc_body)
    return run(m, idx32, zeros)


_BN = 1000
_GRID_C = N_NODES // _BN


def _mlp_body(part_ref, wup_ref, w0_ref, b0_ref, w1_ref, b1_ref,
              w2_ref, b2_ref, wf_ref, out_ref):
    s = part_ref[0] + part_ref[1]
    up = _dot3(s, wup_ref[...])
    for w, b in ((w0_ref, b0_ref), (w1_ref, b1_ref), (w2_ref, b2_ref)):
        up = _dot3(up, w[...]) + b[...]
        up = up * jax.nn.sigmoid(up)
    out_ref[...] = _dot3(up, wf_ref[...])


def _mlp_stage(partials, W_up, W_d0, b_d0, W_d1, b_d1, W_d2, b_d2, W_final):
    wspec = lambda shape: pl.BlockSpec(shape, lambda i: (_i0(),) * len(shape))
    return pl.pallas_call(
        _mlp_body,
        name="mlp_stage",
        grid=(_GRID_C,),
        in_specs=[
            pl.BlockSpec((_NC, _BN, D), lambda i: (_i0(), i, _i0())),
            wspec((D, H)),
            wspec((H, H)), wspec((1, H)),
            wspec((H, H)), wspec((1, H)),
            wspec((H, H)), wspec((1, H)),
            wspec((H, 1)),
        ],
        out_specs=pl.BlockSpec((_BN, 1), lambda i: (i, _i0())),
        out_shape=jax.ShapeDtypeStruct((N_NODES, 1), jnp.float32),
    )(partials, W_up, W_d0, b_d0, W_d1, b_d1, W_d2, b_d2, W_final)


def kernel(messages, rbf, idx_i, n_particles, W_rbf, W_up,
           W_d0, b_d0, W_d1, b_d1, W_d2, b_d2, W_final):
    del n_particles
    out_dtype = jnp.result_type(jnp.float32, W_up.dtype)
    f32 = lambda x: x.astype(jnp.float32)
    idx32 = idx_i.astype(jnp.int32)
    m = _edge_stage(f32(messages), f32(rbf), f32(W_rbf))
    zeros = jnp.zeros((N_NODES, D), jnp.float32)
    partials = _sc_segment(m, idx32, zeros)
    out = _mlp_stage(partials, f32(W_up), f32(W_d0), f32(b_d0).reshape(1, H),
                     f32(W_d1), f32(b_d1).reshape(1, H),
                     f32(W_d2), f32(b_d2).reshape(1, H), f32(W_final))
    return out.astype(out_dtype)
